# baseline (speedup 1.0000x reference)
import functools

import jax
import jax.numpy as jnp
from jax import lax
from jax.experimental import pallas as pl
from jax.experimental.pallas import tpu as pltpu
from jax.experimental.pallas import tpu_sc as plsc

_B = 1024
_D = 64
_V = 100000

_NC = 2
_NS = 16
_NW = _NC * _NS
_B_PER_W = _B // _NW


_NT = 32768
_TG = (_V + _NT - 1) // _NT
_PR = _TG * (_NT // 2)


def _sc_gather(embeddings, center_words):
    table = _pair_table(embeddings)
    mesh = plsc.VectorSubcoreMesh(core_axis_name="c", subcore_axis_name="s")

    @functools.partial(
        pl.kernel,
        out_type=jax.ShapeDtypeStruct((_B, _D), jnp.float32),
        mesh=mesh,
        scratch_types=[
            pltpu.VMEM((_B_PER_W,), jnp.int32),
            pltpu.VMEM((_B_PER_W,), jnp.int32),
            pltpu.VMEM((_B_PER_W, 128), jnp.float32),
            pltpu.VMEM((_B_PER_W, _D), jnp.float32),
            pltpu.SemaphoreType.DMA,
        ],
        compiler_params=pltpu.CompilerParams(skip_device_barrier=True),
    )
    def gather_kernel(table_hbm, idx_hbm, out_hbm, row_v, half_v, rows_v, out_v, sem):
        wid = lax.axis_index("s") * _NC + lax.axis_index("c")
        base = wid * _B_PER_W
        pltpu.sync_copy(idx_hbm.at[pl.ds(base, _B_PER_W)], row_v)
        nt_bits = _NT.bit_length() - 1
        for c in range(_B_PER_W // 16):
            v = row_v[pl.ds(c * 16, 16)]
            g = lax.shift_right_logical(v, nt_bits)
            r = lax.bitwise_and(v, _NT - 1)
            row_v[pl.ds(c * 16, 16)] = (g * (_NT // 2)) + lax.bitwise_and(
                r, _NT // 2 - 1
            )
            half_v[pl.ds(c * 16, 16)] = lax.shift_right_logical(r, nt_bits - 1)
        pltpu.async_copy(table_hbm.at[row_v], rows_v, sem).wait()
        for g in range(_B_PER_W // 16):
            halves = half_v[pl.ds(g * 16, 16)]
            for l in range(16):
                j = g * 16 + l
                take_hi = halves[l] == 1
                for c in range(_D // 16):
                    lo = rows_v[j, pl.ds(c * 16, 16)]
                    hi = rows_v[j, pl.ds(_D + c * 16, 16)]
                    out_v[j, pl.ds(c * 16, 16)] = jnp.where(take_hi, hi, lo)
        pltpu.sync_copy(out_v, out_hbm.at[pl.ds(base, _B_PER_W)])

    return gather_kernel(table, center_words)


def _transpose_body(in_ref, o_ref):
    x = in_ref[...]
    xl = x[:, : _NT // 2]
    xr = x[:, _NT // 2 :]
    o_ref[...] = jnp.concatenate(
        [jnp.transpose(xl, (1, 0)), jnp.transpose(xr, (1, 0))], axis=1
    )


def _pair_table(embeddings):
    return pl.pallas_call(
        _transpose_body,
        grid=(_TG,),
        in_specs=[pl.BlockSpec((_D, _NT), lambda i: (0, i))],
        out_specs=pl.BlockSpec((_NT // 2, 2 * _D), lambda i: (i, 0)),
        out_shape=jax.ShapeDtypeStruct((_PR, 2 * _D), jnp.float32),
    )(embeddings.T)


def _mm_body(w_ref, e_ref, b_ref, o_ref):
    acc = lax.dot_general(
        w_ref[...],
        e_ref[...],
        (((0,), (1,)), ((), ())),
        preferred_element_type=jnp.float32,
    )
    o_ref[...] = acc + jnp.reshape(b_ref[...], (b_ref.shape[0], 1))


def _projection(embedded, W_out, b_out, vt=2048):
    grid = (_V + vt - 1) // vt
    logits_t = pl.pallas_call(
        _mm_body,
        grid=(grid,),
        in_specs=[
            pl.BlockSpec((_D, vt), lambda i: (0, i)),
            pl.BlockSpec((_B, _D), lambda i: (0, 0)),
            pl.BlockSpec((vt,), lambda i: (i,)),
        ],
        out_specs=pl.BlockSpec((vt, _B), lambda i: (i, 0)),
        out_shape=jax.ShapeDtypeStruct((_V, _B), jnp.float32),
    )(W_out.T, embedded, b_out)
    return logits_t.T


def kernel(center_words, embeddings, W_out, b_out):
    embedded = _sc_gather(embeddings, center_words.astype(jnp.int32))
    return _projection(embedded, W_out, b_out)


# speedup vs baseline: 1.0156x; 1.0156x over previous
import functools

import jax
import jax.numpy as jnp
from jax import lax
from jax.experimental import pallas as pl
from jax.experimental.pallas import tpu as pltpu
from jax.experimental.pallas import tpu_sc as plsc

_B = 1024
_D = 64
_V = 100000

_NC = 2
_NS = 16
_NW = _NC * _NS
_B_PER_W = _B // _NW


_NT = 16384
_TG = (_V + _NT - 1) // _NT
_PR = _TG * (_NT // 2)


def _sc_gather(embeddings, center_words):
    table = _pair_table(embeddings)
    mesh = plsc.VectorSubcoreMesh(core_axis_name="c", subcore_axis_name="s")

    @functools.partial(
        pl.kernel,
        out_type=jax.ShapeDtypeStruct((_B, _D), jnp.float32),
        mesh=mesh,
        scratch_types=[
            pltpu.VMEM((_B_PER_W,), jnp.int32),
            pltpu.VMEM((_B_PER_W,), jnp.int32),
            pltpu.VMEM((_B_PER_W, 128), jnp.float32),
            pltpu.VMEM((_B_PER_W, _D), jnp.float32),
            pltpu.SemaphoreType.DMA,
        ],
        compiler_params=pltpu.CompilerParams(skip_device_barrier=True),
    )
    def gather_kernel(table_hbm, idx_hbm, out_hbm, row_v, half_v, rows_v, out_v, sem):
        wid = lax.axis_index("s") * _NC + lax.axis_index("c")
        base = wid * _B_PER_W
        pltpu.sync_copy(idx_hbm.at[pl.ds(base, _B_PER_W)], row_v)
        nt_bits = _NT.bit_length() - 1
        for c in range(_B_PER_W // 16):
            v = row_v[pl.ds(c * 16, 16)]
            g = lax.shift_right_logical(v, nt_bits)
            r = lax.bitwise_and(v, _NT - 1)
            row_v[pl.ds(c * 16, 16)] = (g * (_NT // 2)) + lax.bitwise_and(
                r, _NT // 2 - 1
            )
            half_v[pl.ds(c * 16, 16)] = lax.shift_right_logical(r, nt_bits - 1)
        pltpu.async_copy(table_hbm.at[row_v], rows_v, sem).wait()
        for g in range(_B_PER_W // 16):
            halves = half_v[pl.ds(g * 16, 16)]
            for l in range(16):
                j = g * 16 + l
                take_hi = halves[l] == 1
                for c in range(_D // 16):
                    lo = rows_v[j, pl.ds(c * 16, 16)]
                    hi = rows_v[j, pl.ds(_D + c * 16, 16)]
                    out_v[j, pl.ds(c * 16, 16)] = jnp.where(take_hi, hi, lo)
        pltpu.sync_copy(out_v, out_hbm.at[pl.ds(base, _B_PER_W)])

    return gather_kernel(table, center_words)


def _transpose_body(in_ref, o_ref):
    x = in_ref[...]
    xl = x[:, : _NT // 2]
    xr = x[:, _NT // 2 :]
    o_ref[...] = jnp.concatenate(
        [jnp.transpose(xl, (1, 0)), jnp.transpose(xr, (1, 0))], axis=1
    )


def _pair_table(embeddings):
    return pl.pallas_call(
        _transpose_body,
        grid=(_TG,),
        in_specs=[pl.BlockSpec((_D, _NT), lambda i: (0, i))],
        out_specs=pl.BlockSpec((_NT // 2, 2 * _D), lambda i: (i, 0)),
        out_shape=jax.ShapeDtypeStruct((_PR, 2 * _D), jnp.float32),
        compiler_params=pltpu.CompilerParams(dimension_semantics=("parallel",)),
    )(embeddings.T)


def _mm_body(w_ref, e_ref, b_ref, o_ref):
    acc = lax.dot_general(
        w_ref[...],
        e_ref[...],
        (((0,), (1,)), ((), ())),
        preferred_element_type=jnp.float32,
    )
    o_ref[...] = acc + jnp.reshape(b_ref[...], (b_ref.shape[0], 1))


def _projection(embedded, W_out, b_out, vt=2048):
    grid = (_V + vt - 1) // vt
    logits_t = pl.pallas_call(
        _mm_body,
        grid=(grid,),
        in_specs=[
            pl.BlockSpec((_D, vt), lambda i: (0, i)),
            pl.BlockSpec((_B, _D), lambda i: (0, 0)),
            pl.BlockSpec((vt,), lambda i: (i,)),
        ],
        out_specs=pl.BlockSpec((vt, _B), lambda i: (i, 0)),
        out_shape=jax.ShapeDtypeStruct((_V, _B), jnp.float32),
        compiler_params=pltpu.CompilerParams(dimension_semantics=("parallel",)),
    )(W_out.T, embedded, b_out)
    return logits_t.T


def kernel(center_words, embeddings, W_out, b_out):
    embedded = _sc_gather(embeddings, center_words.astype(jnp.int32))
    return _projection(embedded, W_out, b_out)


# speedup vs baseline: 1.0274x; 1.0117x over previous
import functools

import jax
import jax.numpy as jnp
from jax import lax
from jax.experimental import pallas as pl
from jax.experimental.pallas import tpu as pltpu
from jax.experimental.pallas import tpu_sc as plsc

_B = 1024
_D = 64
_V = 100000

_NC = 2
_NS = 16
_NW = _NC * _NS
_B_PER_W = _B // _NW


_NT = 16384
_TG = (_V + _NT - 1) // _NT
_PR = _TG * (_NT // 2)


def _sc_gather(embeddings, center_words):
    table = _pair_table(embeddings)
    mesh = plsc.VectorSubcoreMesh(core_axis_name="c", subcore_axis_name="s")

    @functools.partial(
        pl.kernel,
        out_type=jax.ShapeDtypeStruct((_B, _D), jnp.float32),
        mesh=mesh,
        scratch_types=[
            pltpu.VMEM((_B_PER_W,), jnp.int32),
            pltpu.VMEM((_B_PER_W,), jnp.int32),
            pltpu.VMEM((_B_PER_W, 128), jnp.float32),
            pltpu.VMEM((_B_PER_W, _D), jnp.float32),
            pltpu.SemaphoreType.DMA,
        ],
        compiler_params=pltpu.CompilerParams(skip_device_barrier=True),
    )
    def gather_kernel(table_hbm, idx_hbm, out_hbm, row_v, half_v, rows_v, out_v, sem):
        wid = lax.axis_index("s") * _NC + lax.axis_index("c")
        base = wid * _B_PER_W
        pltpu.sync_copy(idx_hbm.at[pl.ds(base, _B_PER_W)], row_v)
        nt_bits = _NT.bit_length() - 1
        for c in range(_B_PER_W // 16):
            v = row_v[pl.ds(c * 16, 16)]
            g = lax.shift_right_logical(v, nt_bits)
            r = lax.bitwise_and(v, _NT - 1)
            row_v[pl.ds(c * 16, 16)] = (g * (_NT // 2)) + lax.bitwise_and(
                r, _NT // 2 - 1
            )
            half_v[pl.ds(c * 16, 16)] = lax.shift_right_logical(r, nt_bits - 1)
        pltpu.async_copy(table_hbm.at[row_v], rows_v, sem).wait()
        for g in range(_B_PER_W // 16):
            halves = half_v[pl.ds(g * 16, 16)]
            for l in range(16):
                j = g * 16 + l
                take_hi = halves[l] == 1
                for c in range(_D // 16):
                    lo = rows_v[j, pl.ds(c * 16, 16)]
                    hi = rows_v[j, pl.ds(_D + c * 16, 16)]
                    out_v[j, pl.ds(c * 16, 16)] = jnp.where(take_hi, hi, lo)
        pltpu.sync_copy(out_v, out_hbm.at[pl.ds(base, _B_PER_W)])

    return gather_kernel(table, center_words)


def _transpose_body(in_ref, o_ref):
    x = in_ref[...]
    xl = x[:, : _NT // 2]
    xr = x[:, _NT // 2 :]
    o_ref[...] = jnp.concatenate(
        [jnp.transpose(xl, (1, 0)), jnp.transpose(xr, (1, 0))], axis=1
    )


def _pair_table(embeddings):
    return pl.pallas_call(
        _transpose_body,
        grid=(_TG,),
        in_specs=[pl.BlockSpec((_D, _NT), lambda i: (0, i))],
        out_specs=pl.BlockSpec((_NT // 2, 2 * _D), lambda i: (i, 0)),
        out_shape=jax.ShapeDtypeStruct((_PR, 2 * _D), jnp.float32),
        compiler_params=pltpu.CompilerParams(dimension_semantics=("parallel",)),
    )(embeddings.T)


def _mm_body(w_ref, e_ref, b_ref, o_ref):
    acc = lax.dot_general(
        w_ref[...],
        e_ref[...],
        (((0,), (1,)), ((), ())),
        preferred_element_type=jnp.float32,
    )
    o_ref[...] = acc + jnp.reshape(b_ref[...], (b_ref.shape[0], 1))


def _projection(embedded, W_out, b_out, vt=4096):
    grid = (_V + vt - 1) // vt
    logits_t = pl.pallas_call(
        _mm_body,
        grid=(grid,),
        in_specs=[
            pl.BlockSpec((_D, vt), lambda i: (0, i)),
            pl.BlockSpec((_B, _D), lambda i: (0, 0)),
            pl.BlockSpec((vt,), lambda i: (i,)),
        ],
        out_specs=pl.BlockSpec((vt, _B), lambda i: (i, 0)),
        out_shape=jax.ShapeDtypeStruct((_V, _B), jnp.float32),
        compiler_params=pltpu.CompilerParams(
            dimension_semantics=("parallel",), vmem_limit_bytes=100 * 1024 * 1024
        ),
    )(W_out.T, embedded, b_out)
    return logits_t.T


def kernel(center_words, embeddings, W_out, b_out):
    embedded = _sc_gather(embeddings, center_words.astype(jnp.int32))
    return _projection(embedded, W_out, b_out)


# speedup vs baseline: 1.0315x; 1.0040x over previous
import functools

import jax
import jax.numpy as jnp
from jax import lax
from jax.experimental import pallas as pl
from jax.experimental.pallas import tpu as pltpu
from jax.experimental.pallas import tpu_sc as plsc

_B = 1024
_D = 64
_V = 100000

_NC = 2
_NS = 16
_NW = _NC * _NS
_B_PER_W = _B // _NW


_NT = 16384
_TG = (_V + _NT - 1) // _NT
_PR = _TG * (_NT // 2)


def _sc_gather(embeddings, center_words):
    table = _pair_table(embeddings)
    mesh = plsc.VectorSubcoreMesh(core_axis_name="c", subcore_axis_name="s")

    @functools.partial(
        pl.kernel,
        out_type=jax.ShapeDtypeStruct((_B, _D), jnp.float32),
        mesh=mesh,
        scratch_types=[
            pltpu.VMEM((_B_PER_W,), jnp.int32),
            pltpu.VMEM((_B_PER_W,), jnp.int32),
            pltpu.VMEM((_B_PER_W, 128), jnp.float32),
            pltpu.VMEM((_B_PER_W, _D), jnp.float32),
            pltpu.SemaphoreType.DMA,
        ],
        compiler_params=pltpu.CompilerParams(skip_device_barrier=True),
    )
    def gather_kernel(table_hbm, idx_hbm, out_hbm, row_v, half_v, rows_v, out_v, sem):
        wid = lax.axis_index("s") * _NC + lax.axis_index("c")
        base = wid * _B_PER_W
        pltpu.sync_copy(idx_hbm.at[pl.ds(base, _B_PER_W)], row_v)
        nt_bits = _NT.bit_length() - 1
        for c in range(_B_PER_W // 16):
            v = row_v[pl.ds(c * 16, 16)]
            g = lax.shift_right_logical(v, nt_bits)
            r = lax.bitwise_and(v, _NT - 1)
            row_v[pl.ds(c * 16, 16)] = (g * (_NT // 2)) + lax.bitwise_and(
                r, _NT // 2 - 1
            )
            half_v[pl.ds(c * 16, 16)] = lax.shift_right_logical(r, nt_bits - 1)
        pltpu.async_copy(table_hbm.at[row_v], rows_v, sem).wait()
        for g in range(_B_PER_W // 16):
            halves = half_v[pl.ds(g * 16, 16)]
            for l in range(16):
                j = g * 16 + l
                take_hi = halves[l] == 1
                for c in range(_D // 16):
                    lo = rows_v[j, pl.ds(c * 16, 16)]
                    hi = rows_v[j, pl.ds(_D + c * 16, 16)]
                    out_v[j, pl.ds(c * 16, 16)] = jnp.where(take_hi, hi, lo)
        pltpu.sync_copy(out_v, out_hbm.at[pl.ds(base, _B_PER_W)])

    return gather_kernel(table, center_words)


def _transpose_body(in_ref, o_ref):
    x = in_ref[...]
    xl = x[:, : _NT // 2]
    xr = x[:, _NT // 2 :]
    o_ref[...] = jnp.concatenate(
        [jnp.transpose(xl, (1, 0)), jnp.transpose(xr, (1, 0))], axis=1
    )


def _pair_table(embeddings):
    return pl.pallas_call(
        _transpose_body,
        grid=(_TG,),
        in_specs=[pl.BlockSpec((_D, _NT), lambda i: (0, i))],
        out_specs=pl.BlockSpec((_NT // 2, 2 * _D), lambda i: (i, 0)),
        out_shape=jax.ShapeDtypeStruct((_PR, 2 * _D), jnp.float32),
        compiler_params=pltpu.CompilerParams(dimension_semantics=("parallel",)),
    )(embeddings.T)


def _mm_body(w_ref, e_ref, b_ref, o_ref):
    acc = lax.dot_general(
        w_ref[...],
        e_ref[...],
        (((0,), (1,)), ((), ())),
        preferred_element_type=jnp.float32,
    )
    o_ref[...] = acc + jnp.reshape(b_ref[...], (b_ref.shape[0], 1))


def _projection(embedded, W_out, b_out, vt=6144):
    grid = (_V + vt - 1) // vt
    logits_t = pl.pallas_call(
        _mm_body,
        grid=(grid,),
        in_specs=[
            pl.BlockSpec((_D, vt), lambda i: (0, i)),
            pl.BlockSpec((_B, _D), lambda i: (0, 0)),
            pl.BlockSpec((vt,), lambda i: (i,)),
        ],
        out_specs=pl.BlockSpec((vt, _B), lambda i: (i, 0)),
        out_shape=jax.ShapeDtypeStruct((_V, _B), jnp.float32),
        compiler_params=pltpu.CompilerParams(
            dimension_semantics=("parallel",), vmem_limit_bytes=100 * 1024 * 1024
        ),
    )(W_out.T, embedded, b_out)
    return logits_t.T


def kernel(center_words, embeddings, W_out, b_out):
    embedded = _sc_gather(embeddings, center_words.astype(jnp.int32))
    return _projection(embedded, W_out, b_out)
